# baseline (speedup 1.0000x reference)
import jax
import jax.numpy as jnp
import numpy as np
from jax import lax
from jax.experimental import pallas as pl
from jax.experimental.pallas import tpu as pltpu
from jax.experimental.pallas import tpu_sc as plsc

N = 10000
D = 128
E = 320000

NC = 2
NS = 16
W = 32
NCH = 320
EPS = NCH * W
E_PAD = NC * NS * EPS
PAD_N = 10112
STRIPE = PAD_N // NS
OUT_ROWS = 624
NB = 4
HW = NB // 2


def _sc_agg_body(x_hbm, ei_hbm, z_hbm, out_hbm,
                 idx_v, colr, rowr, gb, agg_sh, *sems):
    c = lax.axis_index("c")
    s = lax.axis_index("s")
    gsems = sems[:NB]
    ssems = sems[NB:]

    pltpu.sync_copy(z_hbm, agg_sh.at[pl.ds(s * STRIPE, STRIPE)])

    ebase = (c * NS + s) * EPS
    pltpu.sync_copy(ei_hbm.at[pl.ds(0, 2), pl.ds(ebase, EPS)], idx_v)

    def unpack(m, ring):
        @pl.loop(0, W, step=16)
        def _(k):
            r = idx_v[0, pl.ds(m * W + k, 16)]
            cc = idx_v[1, pl.ds(m * W + k, 16)]
            colr[ring, pl.ds(k, 16)] = cc
            rowr[ring, pl.ds(k, 16)] = jnp.where(
                r == cc, jnp.int32(N) + (r & jnp.int32(63)), r)

    def start_gather(j, b):
        pltpu.async_copy(x_hbm.at[colr.at[b]], gb.at[b], gsems[b])

    def wait_gather(j, b):
        pltpu.make_async_copy(x_hbm.at[colr.at[b]], gb.at[b],
                              gsems[b]).wait()

    def start_scatter(j, b):
        pltpu.async_copy(gb.at[b], agg_sh.at[rowr.at[b]], ssems[b],
                         add=True)

    def wait_scatter(j, b):
        pltpu.make_async_copy(gb.at[b], agg_sh.at[rowr.at[b]],
                              ssems[b]).wait()

    for m in range(HW):
        unpack(m, m)

    plsc.subcore_barrier()

    for m in range(HW):
        start_gather(m, m)
    for m in range(HW):
        wait_gather(m, m)
        start_scatter(m, m)
        unpack(m + HW, m + HW)
        start_gather(m + HW, m + HW)

    @pl.loop(HW, NCH - HW - (NB - 1), step=NB)
    def _(j):
        for k in range(NB):
            wait_scatter(j + k - HW, k)
            unpack(j + k + HW, k)
            start_gather(j + k + HW, k)
            wait_gather(j + k, (HW + k) % NB)
            start_scatter(j + k, (HW + k) % NB)

    for m in range(NCH - HW, NCH):
        wait_gather(m, m % NB)
        start_scatter(m, m % NB)
    for m in range(NCH - NB, NCH):
        wait_scatter(m, m % NB)

    plsc.subcore_barrier()
    pltpu.sync_copy(agg_sh.at[pl.ds(s * OUT_ROWS, OUT_ROWS)],
                    out_hbm.at[c, pl.ds(s * OUT_ROWS, OUT_ROWS)])

    @pl.when(s == 0)
    def _():
        tail = NS * OUT_ROWS
        pltpu.sync_copy(agg_sh.at[pl.ds(tail, N - tail)],
                        out_hbm.at[c, pl.ds(tail, N - tail)])


def _sc_aggregate(x, ei_p, zeros):
    mesh = plsc.VectorSubcoreMesh(core_axis_name="c", subcore_axis_name="s")
    f = pl.kernel(
        _sc_agg_body,
        out_type=jax.ShapeDtypeStruct((NC, N, D), jnp.float32),
        mesh=mesh,
        scratch_types=[
            pltpu.VMEM((2, EPS), jnp.int32),
            pltpu.VMEM((NB, W), jnp.int32),
            pltpu.VMEM((NB, W), jnp.int32),
            pltpu.VMEM((NB, W, D), jnp.float32),
            pltpu.VMEM_SHARED((PAD_N, D), jnp.float32),
        ] + [pltpu.SemaphoreType.DMA] * (2 * NB),
    )
    return f(x, ei_p, zeros)


def _mlp_body(x_ref, agg_ref, w1_ref, b1_ref, w2_ref, b2_ref, eps_ref, o_ref):
    out = (x_ref[...] * (1.0 + eps_ref[0])
           + agg_ref[0] + agg_ref[1])
    h = jnp.dot(out.astype(jnp.bfloat16), w1_ref[...].astype(jnp.bfloat16),
                preferred_element_type=jnp.float32)
    h = jnp.maximum(h + b1_ref[...], 0.0)
    o_ref[...] = (jnp.dot(h.astype(jnp.bfloat16),
                          w2_ref[...].astype(jnp.bfloat16),
                          preferred_element_type=jnp.float32)
                  + b2_ref[...])


def _mlp(x, agg, W1, b1, W2, b2, eps):
    R = 1000
    grid = (N // R,)
    return pl.pallas_call(
        _mlp_body,
        grid=grid,
        in_specs=[
            pl.BlockSpec((R, D), lambda i: (i, 0)),
            pl.BlockSpec((NC, R, D), lambda i: (0, i, 0)),
            pl.BlockSpec((D, D), lambda i: (0, 0)),
            pl.BlockSpec((1, D), lambda i: (0, 0)),
            pl.BlockSpec((D, D), lambda i: (0, 0)),
            pl.BlockSpec((1, D), lambda i: (0, 0)),
            pl.BlockSpec(memory_space=pltpu.SMEM),
        ],
        out_specs=pl.BlockSpec((R, D), lambda i: (i, 0)),
        out_shape=jax.ShapeDtypeStruct((N, D), jnp.float32),
    )(x, agg, W1, b1.reshape(1, D), W2, b2.reshape(1, D), eps)


_AR = np.arange(E_PAD - E, dtype=np.int64)
_PAD_EDGES = np.stack(
    [N + _AR % (PAD_N - N), _AR % N]).astype(np.int32)


def kernel(x, edge_index, W1, b1, W2, b2, eps):
    ei_p = jnp.concatenate([edge_index, jnp.asarray(_PAD_EDGES)], axis=1)
    zeros = jnp.zeros((STRIPE, D), jnp.float32)
    agg = _sc_aggregate(x, ei_p, zeros)
    return _mlp(x, agg, W1, b1, W2, b2, eps)


# speedup vs baseline: 1.0154x; 1.0154x over previous
import jax
import jax.numpy as jnp
import numpy as np
from jax import lax
from jax.experimental import pallas as pl
from jax.experimental.pallas import tpu as pltpu
from jax.experimental.pallas import tpu_sc as plsc

N = 10000
D = 128
E = 320000

NC = 2
NS = 16
W = 32
NCH = 320
E_PAD = NC * NS * NCH * W
PAD_N = 10112
STRIPE = PAD_N // NS
OUT_ROWS = 624
SHIFT = 14
MASK = (1 << SHIFT) - 1
NB = 8
HW = NB // 2
PK_PER_ROW = 128 // W


def _sc_agg_body(x_hbm, pk_hbm, z_hbm, out_hbm,
                 pk_v, colr, rowr, gb, agg_sh, *sems):
    c = lax.axis_index("c")
    s = lax.axis_index("s")
    gsems = sems[:NB]
    ssems = sems[NB:]

    pltpu.sync_copy(z_hbm, agg_sh.at[pl.ds(s * STRIPE, STRIPE)])

    base = (c * NS + s) * (NCH // PK_PER_ROW)
    pltpu.sync_copy(pk_hbm.at[pl.ds(base, NCH // PK_PER_ROW)], pk_v)

    def unpack(m, ring):
        @pl.loop(0, W, step=16)
        def _(k):
            p = pk_v[m // PK_PER_ROW, pl.ds((m % PK_PER_ROW) * W + k, 16)]
            cc = p & jnp.int32(MASK)
            r = jax.lax.shift_right_logical(p, SHIFT)
            colr[ring, pl.ds(k, 16)] = cc
            rowr[ring, pl.ds(k, 16)] = jnp.where(
                r == cc, jnp.int32(N) + (r & jnp.int32(63)), r)

    def start_gather(j, b):
        pltpu.async_copy(x_hbm.at[colr.at[b]], gb.at[b], gsems[b])

    def wait_gather(j, b):
        pltpu.make_async_copy(x_hbm.at[colr.at[b]], gb.at[b],
                              gsems[b]).wait()

    def start_scatter(j, b):
        pltpu.async_copy(gb.at[b], agg_sh.at[rowr.at[b]], ssems[b],
                         add=True)

    def wait_scatter(j, b):
        pltpu.make_async_copy(gb.at[b], agg_sh.at[rowr.at[b]],
                              ssems[b]).wait()

    for m in range(HW):
        unpack(m, m)

    plsc.subcore_barrier()

    for m in range(HW):
        start_gather(m, m)
    for m in range(HW):
        wait_gather(m, m)
        start_scatter(m, m)
        unpack(m + HW, m + HW)
        start_gather(m + HW, m + HW)

    @pl.loop(HW, NCH - HW - (NB - 1), step=NB)
    def _(j):
        for k in range(NB):
            wait_scatter(j + k - HW, k)
            unpack(j + k + HW, k)
            start_gather(j + k + HW, k)
            wait_gather(j + k, (HW + k) % NB)
            start_scatter(j + k, (HW + k) % NB)

    for m in range(NCH - HW, NCH):
        wait_gather(m, m % NB)
        start_scatter(m, m % NB)
    for m in range(NCH - NB, NCH):
        wait_scatter(m, m % NB)

    plsc.subcore_barrier()
    pltpu.sync_copy(agg_sh.at[pl.ds(s * OUT_ROWS, OUT_ROWS)],
                    out_hbm.at[c, pl.ds(s * OUT_ROWS, OUT_ROWS)])

    @pl.when(s == 0)
    def _():
        tail = NS * OUT_ROWS
        pltpu.sync_copy(agg_sh.at[pl.ds(tail, N - tail)],
                        out_hbm.at[c, pl.ds(tail, N - tail)])


def _sc_aggregate(x, packed, zeros):
    mesh = plsc.VectorSubcoreMesh(core_axis_name="c", subcore_axis_name="s")
    f = pl.kernel(
        _sc_agg_body,
        out_type=jax.ShapeDtypeStruct((NC, N, D), jnp.float32),
        mesh=mesh,
        scratch_types=[
            pltpu.VMEM((NCH // PK_PER_ROW, 128), jnp.int32),
            pltpu.VMEM((NB, W), jnp.int32),
            pltpu.VMEM((NB, W), jnp.int32),
            pltpu.VMEM((NB, W, D), jnp.float32),
            pltpu.VMEM_SHARED((PAD_N, D), jnp.float32),
        ] + [pltpu.SemaphoreType.DMA] * (2 * NB),
    )
    return f(x, packed, zeros)


def _mlp_body(x_ref, agg_ref, w1_ref, b1_ref, w2_ref, b2_ref, eps_ref, o_ref):
    out = (x_ref[...] * (1.0 + eps_ref[0])
           + agg_ref[0] + agg_ref[1])
    h = jnp.dot(out.astype(jnp.bfloat16), w1_ref[...].astype(jnp.bfloat16),
                preferred_element_type=jnp.float32)
    h = jnp.maximum(h + b1_ref[...], 0.0)
    o_ref[...] = (jnp.dot(h.astype(jnp.bfloat16),
                          w2_ref[...].astype(jnp.bfloat16),
                          preferred_element_type=jnp.float32)
                  + b2_ref[...])


def _mlp(x, agg, W1, b1, W2, b2, eps):
    R = 1000
    grid = (N // R,)
    return pl.pallas_call(
        _mlp_body,
        grid=grid,
        in_specs=[
            pl.BlockSpec((R, D), lambda i: (i, 0)),
            pl.BlockSpec((NC, R, D), lambda i: (0, i, 0)),
            pl.BlockSpec((D, D), lambda i: (0, 0)),
            pl.BlockSpec((1, D), lambda i: (0, 0)),
            pl.BlockSpec((D, D), lambda i: (0, 0)),
            pl.BlockSpec((1, D), lambda i: (0, 0)),
            pl.BlockSpec(memory_space=pltpu.SMEM),
        ],
        out_specs=pl.BlockSpec((R, D), lambda i: (i, 0)),
        out_shape=jax.ShapeDtypeStruct((N, D), jnp.float32),
        compiler_params=pltpu.CompilerParams(
            dimension_semantics=("parallel",)),
    )(x, agg, W1, b1.reshape(1, D), W2, b2.reshape(1, D), eps)


_AR = np.arange(E_PAD - E, dtype=np.int64)
_PAD_PACKED = np.asarray(
    ((N + _AR % (PAD_N - N)) << SHIFT) | (_AR % N), dtype=np.int32)


def kernel(x, edge_index, W1, b1, W2, b2, eps):
    packed_real = (edge_index[0] << SHIFT) | edge_index[1]
    packed = jnp.concatenate(
        [packed_real, jnp.asarray(_PAD_PACKED)]).reshape(E_PAD // 128, 128)
    zeros = jnp.zeros((STRIPE, D), jnp.float32)
    agg = _sc_aggregate(x, packed, zeros)
    return _mlp(x, agg, W1, b1, W2, b2, eps)


# speedup vs baseline: 1.0424x; 1.0266x over previous
import jax
import jax.numpy as jnp
import numpy as np
from jax import lax
from jax.experimental import pallas as pl
from jax.experimental.pallas import tpu as pltpu
from jax.experimental.pallas import tpu_sc as plsc

N = 10000
D = 128
E = 320000

NC = 2
NS = 16
W = 32
NCH = 320
E_PAD = NC * NS * NCH * W
PAD_N = 10112
STRIPE = PAD_N // NS
OUT_ROWS = 624
SHIFT = 14
MASK = (1 << SHIFT) - 1
NB = 8
HW = NB // 2
PK_PER_ROW = 128 // W


def _sc_agg_body(x_hbm, pk_hbm, z_hbm, out_hbm,
                 pk_v, colr, rowr, gb, agg_sh, *sems):
    c = lax.axis_index("c")
    s = lax.axis_index("s")
    gsems = sems[:NB]
    ssems = sems[NB:]

    pltpu.sync_copy(z_hbm, agg_sh.at[pl.ds(s * STRIPE, STRIPE)])

    base = (c * NS + s) * (NCH // PK_PER_ROW)
    pltpu.sync_copy(pk_hbm.at[pl.ds(base, NCH // PK_PER_ROW)], pk_v)

    def unpack(m, ring):
        @pl.loop(0, W, step=16)
        def _(k):
            p = pk_v[m // PK_PER_ROW, pl.ds((m % PK_PER_ROW) * W + k, 16)]
            cc = p & jnp.int32(MASK)
            r = jax.lax.shift_right_logical(p, SHIFT)
            colr[ring, pl.ds(k, 16)] = cc
            rowr[ring, pl.ds(k, 16)] = jnp.where(
                r == cc, jnp.int32(N) + (r & jnp.int32(63)), r)

    def start_gather(j, b):
        pltpu.async_copy(x_hbm.at[colr.at[b]], gb.at[b], gsems[b])

    def wait_gather(j, b):
        pltpu.make_async_copy(x_hbm.at[colr.at[b]], gb.at[b],
                              gsems[b]).wait()

    def start_scatter(j, b):
        pltpu.async_copy(gb.at[b], agg_sh.at[rowr.at[b]], ssems[b],
                         add=True)

    def wait_scatter(j, b):
        pltpu.make_async_copy(gb.at[b], agg_sh.at[rowr.at[b]],
                              ssems[b]).wait()

    for m in range(HW):
        unpack(m, m)

    plsc.subcore_barrier()

    for m in range(HW):
        start_gather(m, m)
    for m in range(HW):
        wait_gather(m, m)
        start_scatter(m, m)
        unpack(m + HW, m + HW)
        start_gather(m + HW, m + HW)

    @pl.loop(HW, NCH - HW - (NB - 1), step=NB)
    def _(j):
        for k in range(NB):
            wait_scatter(j + k - HW, k)
            unpack(j + k + HW, k)
            start_gather(j + k + HW, k)
            wait_gather(j + k, (HW + k) % NB)
            start_scatter(j + k, (HW + k) % NB)

    for m in range(NCH - HW, NCH):
        wait_gather(m, m % NB)
        start_scatter(m, m % NB)
    for m in range(NCH - NB, NCH):
        wait_scatter(m, m % NB)

    plsc.subcore_barrier()
    pltpu.sync_copy(agg_sh.at[pl.ds(s * OUT_ROWS, OUT_ROWS)],
                    out_hbm.at[c, pl.ds(s * OUT_ROWS, OUT_ROWS)])

    @pl.when(s == 0)
    def _():
        tail = NS * OUT_ROWS
        pltpu.sync_copy(agg_sh.at[pl.ds(tail, N - tail)],
                        out_hbm.at[c, pl.ds(tail, N - tail)])


def _sc_aggregate(x, packed, zeros):
    mesh = plsc.VectorSubcoreMesh(core_axis_name="c", subcore_axis_name="s")
    f = pl.kernel(
        _sc_agg_body,
        out_type=jax.ShapeDtypeStruct((NC, N, D), jnp.float32),
        mesh=mesh,
        scratch_types=[
            pltpu.VMEM((NCH // PK_PER_ROW, 128), jnp.int32),
            pltpu.VMEM((NB, W), jnp.int32),
            pltpu.VMEM((NB, W), jnp.int32),
            pltpu.VMEM((NB, W, D), jnp.float32),
            pltpu.VMEM_SHARED((PAD_N, D), jnp.float32),
        ] + [pltpu.SemaphoreType.DMA] * (2 * NB),
    )
    return f(x, packed, zeros)


def _mlp_body(x_ref, agg_ref, w1_ref, b1_ref, w2_ref, b2_ref, eps_ref, o_ref):
    out = (x_ref[...] * (1.0 + eps_ref[0])
           + agg_ref[0] + agg_ref[1])
    h = jnp.dot(out.astype(jnp.bfloat16), w1_ref[...].astype(jnp.bfloat16),
                preferred_element_type=jnp.float32)
    h = jnp.maximum(h + b1_ref[...], 0.0)
    o_ref[...] = (jnp.dot(h.astype(jnp.bfloat16),
                          w2_ref[...].astype(jnp.bfloat16),
                          preferred_element_type=jnp.float32)
                  + b2_ref[...])


def _mlp(x, agg, W1, b1, W2, b2, eps):
    R = 1000
    grid = (N // R,)
    return pl.pallas_call(
        _mlp_body,
        grid=grid,
        in_specs=[
            pl.BlockSpec((R, D), lambda i: (i, 0)),
            pl.BlockSpec((NC, R, D), lambda i: (0, i, 0)),
            pl.BlockSpec((D, D), lambda i: (0, 0)),
            pl.BlockSpec((1, D), lambda i: (0, 0)),
            pl.BlockSpec((D, D), lambda i: (0, 0)),
            pl.BlockSpec((1, D), lambda i: (0, 0)),
            pl.BlockSpec(memory_space=pltpu.SMEM),
        ],
        out_specs=pl.BlockSpec((R, D), lambda i: (i, 0)),
        out_shape=jax.ShapeDtypeStruct((N, D), jnp.float32),
        compiler_params=pltpu.CompilerParams(
            dimension_semantics=("parallel",)),
    )(x, agg, W1, b1.reshape(1, D), W2, b2.reshape(1, D), eps)


_AR = np.arange(E_PAD - E, dtype=np.int64)
_PAD_EDGES = np.stack(
    [N + _AR % (PAD_N - N), _AR % N]).astype(np.int32)


def _pack_body(ei_ref, o_ref):
    blk = o_ref.shape[0]
    r = ei_ref[0:1, :].reshape(blk, 128)
    c = ei_ref[1:2, :].reshape(blk, 128)
    o_ref[...] = (r << SHIFT) | c


def _pack(ei_p):
    G = 8
    rows = E_PAD // 128 // G
    return pl.pallas_call(
        _pack_body,
        grid=(G,),
        in_specs=[pl.BlockSpec((2, rows * 128), lambda i: (0, i))],
        out_specs=pl.BlockSpec((rows, 128), lambda i: (i, 0)),
        out_shape=jax.ShapeDtypeStruct((E_PAD // 128, 128), jnp.int32),
        compiler_params=pltpu.CompilerParams(
            dimension_semantics=("arbitrary",)),
    )(ei_p)


def kernel(x, edge_index, W1, b1, W2, b2, eps):
    ei_p = jnp.concatenate([edge_index, jnp.asarray(_PAD_EDGES)], axis=1)
    packed = _pack(ei_p)
    zeros = jnp.zeros((STRIPE, D), jnp.float32)
    agg = _sc_aggregate(x, packed, zeros)
    return _mlp(x, agg, W1, b1, W2, b2, eps)


# speedup vs baseline: 1.0578x; 1.0147x over previous
import jax
import jax.numpy as jnp
import numpy as np
from jax import lax
from jax.experimental import pallas as pl
from jax.experimental.pallas import tpu as pltpu
from jax.experimental.pallas import tpu_sc as plsc

N = 10000
D = 128
E = 320000

NC = 2
NS = 16
W = 32
NCH = 320
E_PAD = NC * NS * NCH * W
PAD_N = 10112
STRIPE = PAD_N // NS
OUT_ROWS = 624
SHIFT = 14
MASK = (1 << SHIFT) - 1
NB = 8
HW = NB // 2
PK_PER_ROW = 128 // W


def _sc_agg_body(x_hbm, pk_hbm, z_hbm, out_hbm,
                 pk_v, colr, rowr, gb, agg_sh, *sems):
    c = lax.axis_index("c")
    s = lax.axis_index("s")
    gsems = sems[:NB]
    ssems = sems[NB:]

    pltpu.sync_copy(z_hbm, agg_sh.at[pl.ds(s * STRIPE, STRIPE)])

    base = (c * NS + s) * (NCH // PK_PER_ROW)
    pltpu.sync_copy(pk_hbm.at[pl.ds(base, NCH // PK_PER_ROW)], pk_v)

    def unpack(m, ring):
        @pl.loop(0, W, step=16)
        def _(k):
            p = pk_v[m // PK_PER_ROW, pl.ds((m % PK_PER_ROW) * W + k, 16)]
            cc = p & jnp.int32(MASK)
            r = jax.lax.shift_right_logical(p, SHIFT)
            colr[ring, pl.ds(k, 16)] = cc
            rowr[ring, pl.ds(k, 16)] = jnp.where(
                r == cc, jnp.int32(N) + (r & jnp.int32(63)), r)

    def start_gather(j, b):
        pltpu.async_copy(x_hbm.at[colr.at[b]], gb.at[b], gsems[b])

    def wait_gather(j, b):
        pltpu.make_async_copy(x_hbm.at[colr.at[b]], gb.at[b],
                              gsems[b]).wait()

    def start_scatter(j, b):
        pltpu.async_copy(gb.at[b], agg_sh.at[rowr.at[b]], ssems[b],
                         add=True)

    def wait_scatter(j, b):
        pltpu.make_async_copy(gb.at[b], agg_sh.at[rowr.at[b]],
                              ssems[b]).wait()

    for m in range(HW):
        unpack(m, m)

    plsc.subcore_barrier()

    for m in range(HW):
        start_gather(m, m)
    for m in range(HW):
        wait_gather(m, m)
        start_scatter(m, m)
        unpack(m + HW, m + HW)
        start_gather(m + HW, m + HW)

    @pl.loop(HW, NCH - HW - (NB - 1), step=NB)
    def _(j):
        for k in range(NB):
            wait_scatter(j + k - HW, k)
            unpack(j + k + HW, k)
            start_gather(j + k + HW, k)
            wait_gather(j + k, (HW + k) % NB)
            start_scatter(j + k, (HW + k) % NB)

    for m in range(NCH - HW, NCH):
        wait_gather(m, m % NB)
        start_scatter(m, m % NB)
    for m in range(NCH - NB, NCH):
        wait_scatter(m, m % NB)

    plsc.subcore_barrier()
    pltpu.sync_copy(agg_sh.at[pl.ds(s * OUT_ROWS, OUT_ROWS)],
                    out_hbm.at[c, pl.ds(s * OUT_ROWS, OUT_ROWS)])

    @pl.when(s == 0)
    def _():
        tail = NS * OUT_ROWS
        pltpu.sync_copy(agg_sh.at[pl.ds(tail, N - tail)],
                        out_hbm.at[c, pl.ds(tail, N - tail)])


def _sc_aggregate(x, packed, zeros):
    mesh = plsc.VectorSubcoreMesh(core_axis_name="c", subcore_axis_name="s")
    f = pl.kernel(
        _sc_agg_body,
        out_type=jax.ShapeDtypeStruct((NC, N, D), jnp.float32),
        mesh=mesh,
        scratch_types=[
            pltpu.VMEM((NCH // PK_PER_ROW, 128), jnp.int32),
            pltpu.VMEM((NB, W), jnp.int32),
            pltpu.VMEM((NB, W), jnp.int32),
            pltpu.VMEM((NB, W, D), jnp.float32),
            pltpu.VMEM_SHARED((PAD_N, D), jnp.float32),
        ] + [pltpu.SemaphoreType.DMA] * (2 * NB),
    )
    return f(x, packed, zeros)


def _mlp_body(x_ref, agg_ref, w1_ref, b1_ref, w2_ref, b2_ref, eps_ref, o_ref):
    out = (x_ref[...] * (1.0 + eps_ref[0])
           + agg_ref[0] + agg_ref[1])
    h = jnp.dot(out.astype(jnp.bfloat16), w1_ref[...].astype(jnp.bfloat16),
                preferred_element_type=jnp.float32)
    h = jnp.maximum(h + b1_ref[...], 0.0)
    o_ref[...] = (jnp.dot(h.astype(jnp.bfloat16),
                          w2_ref[...].astype(jnp.bfloat16),
                          preferred_element_type=jnp.float32)
                  + b2_ref[...])


def _mlp(x, agg, W1, b1, W2, b2, eps):
    R = 2000
    grid = (N // R,)
    return pl.pallas_call(
        _mlp_body,
        grid=grid,
        in_specs=[
            pl.BlockSpec((R, D), lambda i: (i, 0)),
            pl.BlockSpec((NC, R, D), lambda i: (0, i, 0)),
            pl.BlockSpec((D, D), lambda i: (0, 0)),
            pl.BlockSpec((1, D), lambda i: (0, 0)),
            pl.BlockSpec((D, D), lambda i: (0, 0)),
            pl.BlockSpec((1, D), lambda i: (0, 0)),
            pl.BlockSpec(memory_space=pltpu.SMEM),
        ],
        out_specs=pl.BlockSpec((R, D), lambda i: (i, 0)),
        out_shape=jax.ShapeDtypeStruct((N, D), jnp.float32),
        compiler_params=pltpu.CompilerParams(
            dimension_semantics=("parallel",)),
    )(x, agg, W1, b1.reshape(1, D), W2, b2.reshape(1, D), eps)


_AR = np.arange(E_PAD - E, dtype=np.int64)
_PAD_EDGES = np.stack(
    [N + _AR % (PAD_N - N), _AR % N]).astype(np.int32)


def _pack_body(ei_ref, o_ref):
    blk = o_ref.shape[0]
    r = ei_ref[0:1, :].reshape(blk, 128)
    c = ei_ref[1:2, :].reshape(blk, 128)
    o_ref[...] = (r << SHIFT) | c


def _pack(ei_p):
    G = 8
    rows = E_PAD // 128 // G
    return pl.pallas_call(
        _pack_body,
        grid=(G,),
        in_specs=[pl.BlockSpec((2, rows * 128), lambda i: (0, i))],
        out_specs=pl.BlockSpec((rows, 128), lambda i: (i, 0)),
        out_shape=jax.ShapeDtypeStruct((E_PAD // 128, 128), jnp.int32),
        compiler_params=pltpu.CompilerParams(
            dimension_semantics=("arbitrary",)),
    )(ei_p)


def kernel(x, edge_index, W1, b1, W2, b2, eps):
    ei_p = jnp.concatenate([edge_index, jnp.asarray(_PAD_EDGES)], axis=1)
    packed = _pack(ei_p)
    zeros = jnp.zeros((STRIPE, D), jnp.float32)
    agg = _sc_aggregate(x, packed, zeros)
    return _mlp(x, agg, W1, b1, W2, b2, eps)
